# baseline (speedup 1.0000x reference)
import functools

import jax
import jax.numpy as jnp
from jax import lax
from jax.experimental import pallas as pl
from jax.experimental.pallas import tpu as pltpu
from jax.experimental.pallas import tpu_sc as plsc

B = 16384
EMB = 32
HID = 32
N_USERS = 1000000
N_MOVIES = 100000
NC = 2
NS = 16
NW = NC * NS
BPW = B // NW
CHUNK = 128
NCHUNK = BPW // CHUNK
BLKC = 32768
GRP = BLKC // 4


def _tc_transpose_pack(table_t, n_rows):
    nblk = (n_rows + BLKC - 1) // BLKC

    def body(x_ref, o_ref):
        for k in range(4):
            o_ref[:, k * EMB:(k + 1) * EMB] = x_ref[:, k * GRP:(k + 1) * GRP].T

    return pl.pallas_call(
        body,
        grid=(nblk,),
        in_specs=[pl.BlockSpec((EMB, BLKC), lambda i: (0, i))],
        out_specs=pl.BlockSpec((GRP, 128), lambda i: (i, 0)),
        out_shape=jax.ShapeDtypeStruct((nblk * GRP, 128), jnp.float32),
    )(table_t)


def _sc_gather(u_idx2d, m_idx2d, ut_p, mt_p):
    mesh = plsc.VectorSubcoreMesh(core_axis_name="c", subcore_axis_name="s")

    @functools.partial(
        pl.kernel,
        mesh=mesh,
        out_type=(
            jax.ShapeDtypeStruct((B, 128), jnp.float32),
            jax.ShapeDtypeStruct((B, 128), jnp.float32),
        ),
        scratch_types=[
            pltpu.VMEM((NCHUNK, CHUNK), jnp.int32),
            pltpu.VMEM((NCHUNK, CHUNK), jnp.int32),
            pltpu.VMEM((NCHUNK, CHUNK), jnp.int32),
            pltpu.VMEM((NCHUNK, CHUNK), jnp.int32),
            pltpu.VMEM((CHUNK, 128), jnp.float32),
            pltpu.VMEM((CHUNK, 128), jnp.float32),
            pltpu.SemaphoreType.DMA,
            pltpu.SemaphoreType.DMA,
        ],
    )
    def k(u_idx_hbm, m_idx_hbm, ut_hbm, mt_hbm, u_out, m_out,
          uidx_v, midx_v, uln_v, mln_v, ubuf_v, mbuf_v, sem_u, sem_m):
        wid = lax.axis_index("s") * NC + lax.axis_index("c")
        base = wid * BPW
        pltpu.sync_copy(u_idx_hbm.at[pl.ds(wid * NCHUNK, NCHUNK)], uidx_v)
        pltpu.sync_copy(m_idx_hbm.at[pl.ds(wid * NCHUNK, NCHUNK)], midx_v)
        L = 16
        for j in range(NCHUNK):
            for g in range(CHUNK // L):
                sl = pl.ds(g * L, L)
                uv = uidx_v[j, sl]
                mv = midx_v[j, sl]
                uln_v[j, sl] = (lax.shift_left(
                    lax.shift_right_logical(uv, 15), 13)
                    + lax.bitwise_and(uv, GRP - 1))
                mln_v[j, sl] = (lax.shift_left(
                    lax.shift_right_logical(mv, 15), 13)
                    + lax.bitwise_and(mv, GRP - 1))
        for j in range(NCHUNK):
            cu = pltpu.async_copy(ut_hbm.at[uln_v.at[j]], ubuf_v, sem_u)
            cm = pltpu.async_copy(mt_hbm.at[mln_v.at[j]], mbuf_v, sem_m)
            cu.wait()
            pltpu.sync_copy(ubuf_v, u_out.at[pl.ds(base + j * CHUNK, CHUNK)])
            cm.wait()
            pltpu.sync_copy(mbuf_v, m_out.at[pl.ds(base + j * CHUNK, CHUNK)])

    return k(u_idx2d, m_idx2d, ut_p, mt_p)


def _tc_mlp(u_rows, m_rows, ku, km, w1u_t, w1m_t, b1_2d, w2_2d, b2_2d):
    BLK = 2048

    def body(u_ref, m_ref, ku_ref, km_ref, w1u_ref, w1m_ref, b1_ref,
             w2_ref, b2_ref, o_ref):
        xu = jnp.zeros((BLK, EMB), jnp.float32)
        xm = jnp.zeros((BLK, EMB), jnp.float32)
        kub = ku_ref[...]
        kmb = km_ref[...]
        for k in range(4):
            su = (kub == k).astype(jnp.float32)
            sm = (kmb == k).astype(jnp.float32)
            xu = xu + su * u_ref[:, k * EMB:(k + 1) * EMB]
            xm = xm + sm * m_ref[:, k * EMB:(k + 1) * EMB]
        h = jnp.dot(xu, w1u_ref[...], preferred_element_type=jnp.float32)
        h = h + jnp.dot(xm, w1m_ref[...], preferred_element_type=jnp.float32)
        h = jnp.maximum(h + b1_ref[...], 0.0)
        o = jnp.sum(h * w2_ref[...], axis=1, keepdims=True) + b2_ref[0, 0]
        o_ref[...] = jnp.maximum(o, 0.0)

    out = pl.pallas_call(
        body,
        grid=(B // BLK,),
        in_specs=[
            pl.BlockSpec((BLK, 128), lambda i: (i, 0)),
            pl.BlockSpec((BLK, 128), lambda i: (i, 0)),
            pl.BlockSpec((BLK, 1), lambda i: (i, 0)),
            pl.BlockSpec((BLK, 1), lambda i: (i, 0)),
            pl.BlockSpec((EMB, HID), lambda i: (0, 0)),
            pl.BlockSpec((EMB, HID), lambda i: (0, 0)),
            pl.BlockSpec((1, HID), lambda i: (0, 0)),
            pl.BlockSpec((1, HID), lambda i: (0, 0)),
            pl.BlockSpec((1, 1), lambda i: (0, 0)),
        ],
        out_specs=pl.BlockSpec((BLK, 1), lambda i: (i, 0)),
        out_shape=jax.ShapeDtypeStruct((B, 1), jnp.float32),
    )(u_rows, m_rows, ku, km, w1u_t, w1m_t, b1_2d, w2_2d, b2_2d)
    return out[:, 0]


def kernel(u_idx, m_idx, user_table, movie_table, W1, b1, W2, b2):
    u32 = u_idx.astype(jnp.int32)
    m32 = m_idx.astype(jnp.int32)
    u_idx2d = u32.reshape(B // CHUNK, CHUNK)
    m_idx2d = m32.reshape(B // CHUNK, CHUNK)
    ut_p = _tc_transpose_pack(user_table.T, N_USERS)
    mt_p = _tc_transpose_pack(movie_table.T, N_MOVIES)
    u_rows, m_rows = _sc_gather(u_idx2d, m_idx2d, ut_p, mt_p)
    ku = lax.bitwise_and(lax.shift_right_logical(u32, 13), 3).reshape(B, 1)
    km = lax.bitwise_and(lax.shift_right_logical(m32, 13), 3).reshape(B, 1)
    w1u_t = W1[:, :EMB].T
    w1m_t = W1[:, EMB:].T
    return _tc_mlp(u_rows, m_rows, ku, km, w1u_t, w1m_t,
                   b1.reshape(1, HID), W2, b2.reshape(1, 1))


# speedup vs baseline: 1.0023x; 1.0023x over previous
import functools

import jax
import jax.numpy as jnp
from jax import lax
from jax.experimental import pallas as pl
from jax.experimental.pallas import tpu as pltpu
from jax.experimental.pallas import tpu_sc as plsc

B = 16384
EMB = 32
HID = 32
N_USERS = 1000000
N_MOVIES = 100000
NC = 2
NS = 16
NW = NC * NS
BPW = B // NW
CHUNK = 128
NCHUNK = BPW // CHUNK
BLKC = 8192
GRP = BLKC // 4


def _tc_transpose_pack(table_t, n_rows):
    nblk = (n_rows + BLKC - 1) // BLKC

    def body(x_ref, o_ref):
        for k in range(4):
            o_ref[:, k * EMB:(k + 1) * EMB] = x_ref[:, k * GRP:(k + 1) * GRP].T

    return pl.pallas_call(
        body,
        grid=(nblk,),
        in_specs=[pl.BlockSpec((EMB, BLKC), lambda i: (0, i))],
        out_specs=pl.BlockSpec((GRP, 128), lambda i: (i, 0)),
        out_shape=jax.ShapeDtypeStruct((nblk * GRP, 128), jnp.float32),
    )(table_t)


def _sc_gather(u_idx2d, m_idx2d, ut_p, mt_p):
    mesh = plsc.VectorSubcoreMesh(core_axis_name="c", subcore_axis_name="s")

    @functools.partial(
        pl.kernel,
        mesh=mesh,
        out_type=(
            jax.ShapeDtypeStruct((B, 128), jnp.float32),
            jax.ShapeDtypeStruct((B, 128), jnp.float32),
        ),
        scratch_types=[
            pltpu.VMEM((NCHUNK, CHUNK), jnp.int32),
            pltpu.VMEM((NCHUNK, CHUNK), jnp.int32),
            pltpu.VMEM((NCHUNK, CHUNK), jnp.int32),
            pltpu.VMEM((NCHUNK, CHUNK), jnp.int32),
            pltpu.VMEM((CHUNK, 128), jnp.float32),
            pltpu.VMEM((CHUNK, 128), jnp.float32),
            pltpu.SemaphoreType.DMA,
            pltpu.SemaphoreType.DMA,
        ],
    )
    def k(u_idx_hbm, m_idx_hbm, ut_hbm, mt_hbm, u_out, m_out,
          uidx_v, midx_v, uln_v, mln_v, ubuf_v, mbuf_v, sem_u, sem_m):
        wid = lax.axis_index("s") * NC + lax.axis_index("c")
        base = wid * BPW
        pltpu.sync_copy(u_idx_hbm.at[pl.ds(wid * NCHUNK, NCHUNK)], uidx_v)
        pltpu.sync_copy(m_idx_hbm.at[pl.ds(wid * NCHUNK, NCHUNK)], midx_v)
        L = 16
        for j in range(NCHUNK):
            for g in range(CHUNK // L):
                sl = pl.ds(g * L, L)
                uv = uidx_v[j, sl]
                mv = midx_v[j, sl]
                uln_v[j, sl] = (lax.shift_left(
                    lax.shift_right_logical(uv, 13), 11)
                    + lax.bitwise_and(uv, GRP - 1))
                mln_v[j, sl] = (lax.shift_left(
                    lax.shift_right_logical(mv, 13), 11)
                    + lax.bitwise_and(mv, GRP - 1))
        for j in range(NCHUNK):
            cu = pltpu.async_copy(ut_hbm.at[uln_v.at[j]], ubuf_v, sem_u)
            cm = pltpu.async_copy(mt_hbm.at[mln_v.at[j]], mbuf_v, sem_m)
            cu.wait()
            pltpu.sync_copy(ubuf_v, u_out.at[pl.ds(base + j * CHUNK, CHUNK)])
            cm.wait()
            pltpu.sync_copy(mbuf_v, m_out.at[pl.ds(base + j * CHUNK, CHUNK)])

    return k(u_idx2d, m_idx2d, ut_p, mt_p)


def _tc_mlp(u_rows, m_rows, ku, km, w1u_t, w1m_t, b1_2d, w2_2d, b2_2d):
    BLK = 2048

    def body(u_ref, m_ref, ku_ref, km_ref, w1u_ref, w1m_ref, b1_ref,
             w2_ref, b2_ref, o_ref):
        xu = jnp.zeros((BLK, EMB), jnp.float32)
        xm = jnp.zeros((BLK, EMB), jnp.float32)
        kub = ku_ref[...]
        kmb = km_ref[...]
        for k in range(4):
            su = (kub == k).astype(jnp.float32)
            sm = (kmb == k).astype(jnp.float32)
            xu = xu + su * u_ref[:, k * EMB:(k + 1) * EMB]
            xm = xm + sm * m_ref[:, k * EMB:(k + 1) * EMB]
        h = jnp.dot(xu, w1u_ref[...], preferred_element_type=jnp.float32)
        h = h + jnp.dot(xm, w1m_ref[...], preferred_element_type=jnp.float32)
        h = jnp.maximum(h + b1_ref[...], 0.0)
        o = jnp.sum(h * w2_ref[...], axis=1, keepdims=True) + b2_ref[0, 0]
        o_ref[...] = jnp.maximum(o, 0.0)

    out = pl.pallas_call(
        body,
        grid=(B // BLK,),
        in_specs=[
            pl.BlockSpec((BLK, 128), lambda i: (i, 0)),
            pl.BlockSpec((BLK, 128), lambda i: (i, 0)),
            pl.BlockSpec((BLK, 1), lambda i: (i, 0)),
            pl.BlockSpec((BLK, 1), lambda i: (i, 0)),
            pl.BlockSpec((EMB, HID), lambda i: (0, 0)),
            pl.BlockSpec((EMB, HID), lambda i: (0, 0)),
            pl.BlockSpec((1, HID), lambda i: (0, 0)),
            pl.BlockSpec((1, HID), lambda i: (0, 0)),
            pl.BlockSpec((1, 1), lambda i: (0, 0)),
        ],
        out_specs=pl.BlockSpec((BLK, 1), lambda i: (i, 0)),
        out_shape=jax.ShapeDtypeStruct((B, 1), jnp.float32),
    )(u_rows, m_rows, ku, km, w1u_t, w1m_t, b1_2d, w2_2d, b2_2d)
    return out[:, 0]


def kernel(u_idx, m_idx, user_table, movie_table, W1, b1, W2, b2):
    u32 = u_idx.astype(jnp.int32)
    m32 = m_idx.astype(jnp.int32)
    u_idx2d = u32.reshape(B // CHUNK, CHUNK)
    m_idx2d = m32.reshape(B // CHUNK, CHUNK)
    ut_p = _tc_transpose_pack(user_table.T, N_USERS)
    mt_p = _tc_transpose_pack(movie_table.T, N_MOVIES)
    u_rows, m_rows = _sc_gather(u_idx2d, m_idx2d, ut_p, mt_p)
    ku = lax.bitwise_and(lax.shift_right_logical(u32, 11), 3).reshape(B, 1)
    km = lax.bitwise_and(lax.shift_right_logical(m32, 11), 3).reshape(B, 1)
    w1u_t = W1[:, :EMB].T
    w1m_t = W1[:, EMB:].T
    return _tc_mlp(u_rows, m_rows, ku, km, w1u_t, w1m_t,
                   b1.reshape(1, HID), W2, b2.reshape(1, 1))
